# Initial kernel scaffold; baseline (speedup 1.0000x reference)
import jax
import jax.numpy as jnp
from jax.experimental import pallas as pl


def kernel(x, task_id, table, W1, b1, W2, b2):
    raise NotImplementedError("write your pallas kernel here")


# speedup vs baseline: 1.3364x; 1.3364x over previous
import jax
import jax.numpy as jnp
from jax.experimental import pallas as pl
from jax.experimental.pallas import tpu as pltpu

D = 4096
NUM_TASKS = 16
NUM_EXPERTS = 64
TOP_K = 8
TOKENS = 8192

T_BLK = 1024
N_BLK = 512


def _proj_kernel(table_ref, w1b_ref, b1_ref, out_ref):
    out_ref[...] = (
        jnp.dot(table_ref[...], w1b_ref[...], preferred_element_type=jnp.float32)
        + b1_ref[...]
    )


def _router_kernel(x_ref, tid_ref, w1t_ref, tp_ref, w2_ref, b2_ref,
                   idx_ref, w_ref, logits_acc):
    n = pl.program_id(1)
    nblocks = pl.num_programs(1)

    h = jnp.dot(x_ref[...], w1t_ref[...], preferred_element_type=jnp.float32)
    oh = (tid_ref[...] == jax.lax.broadcasted_iota(
        jnp.int32, (T_BLK, NUM_TASKS), 1)).astype(jnp.float32)
    h = h + jnp.dot(oh, tp_ref[...], preferred_element_type=jnp.float32,
                    precision=jax.lax.Precision.HIGHEST)
    h = jnp.maximum(h, 0.0)
    partial = jnp.dot(h.astype(jnp.bfloat16), w2_ref[...],
                      preferred_element_type=jnp.float32)

    @pl.when(n == 0)
    def _():
        logits_acc[...] = partial

    @pl.when(n > 0)
    def _():
        logits_acc[...] += partial

    @pl.when(n == nblocks - 1)
    def _():
        logits = logits_acc[...] + b2_ref[...]
        iota = jax.lax.broadcasted_iota(jnp.int32, (T_BLK, NUM_EXPERTS), 1)
        vals = logits
        neg_inf = jnp.float32(-jnp.inf)
        top_v = []
        top_i = []
        for _k in range(TOP_K):
            m = jnp.max(vals, axis=1, keepdims=True)
            is_max = vals == m
            idx = jnp.min(jnp.where(is_max, iota, NUM_EXPERTS), axis=1,
                          keepdims=True)
            top_v.append(m)
            top_i.append(idx)
            vals = jnp.where(iota == idx, neg_inf, vals)
        tv = jnp.concatenate(top_v, axis=1)
        ti = jnp.concatenate(top_i, axis=1)
        e = jnp.exp(tv - tv[:, 0:1])
        w = e / jnp.sum(e, axis=1, keepdims=True)
        idx_ref[...] = ti
        w_ref[...] = w


def kernel(x, task_id, table, W1, b1, W2, b2):
    x_bf = x.astype(jnp.bfloat16)
    w1_top = W1[:D].astype(jnp.bfloat16)
    w1_bot = W1[D:].astype(jnp.bfloat16)
    table_bf = table.astype(jnp.bfloat16)
    w2_bf = W2.astype(jnp.bfloat16)
    b1_2d = b1.reshape(1, D)
    b2_2d = b2.reshape(1, NUM_EXPERTS)
    tid_2d = task_id.astype(jnp.int32).reshape(TOKENS, 1)

    n_nb = D // N_BLK
    table_proj = pl.pallas_call(
        _proj_kernel,
        grid=(n_nb,),
        in_specs=[
            pl.BlockSpec((NUM_TASKS, D), lambda n: (0, 0)),
            pl.BlockSpec((D, N_BLK), lambda n: (0, n)),
            pl.BlockSpec((1, N_BLK), lambda n: (0, n)),
        ],
        out_specs=pl.BlockSpec((NUM_TASKS, N_BLK), lambda n: (0, n)),
        out_shape=jax.ShapeDtypeStruct((NUM_TASKS, D), jnp.float32),
    )(table_bf, w1_bot, b1_2d)

    n_tb = TOKENS // T_BLK
    indices, weights = pl.pallas_call(
        _router_kernel,
        grid=(n_tb, n_nb),
        in_specs=[
            pl.BlockSpec((T_BLK, D), lambda t, n: (t, 0)),
            pl.BlockSpec((T_BLK, 1), lambda t, n: (t, 0)),
            pl.BlockSpec((D, N_BLK), lambda t, n: (0, n)),
            pl.BlockSpec((NUM_TASKS, N_BLK), lambda t, n: (0, n)),
            pl.BlockSpec((N_BLK, NUM_EXPERTS), lambda t, n: (n, 0)),
            pl.BlockSpec((1, NUM_EXPERTS), lambda t, n: (0, 0)),
        ],
        out_specs=[
            pl.BlockSpec((T_BLK, TOP_K), lambda t, n: (t, 0)),
            pl.BlockSpec((T_BLK, TOP_K), lambda t, n: (t, 0)),
        ],
        out_shape=[
            jax.ShapeDtypeStruct((TOKENS, TOP_K), jnp.int32),
            jax.ShapeDtypeStruct((TOKENS, TOP_K), jnp.float32),
        ],
        scratch_shapes=[pltpu.VMEM((T_BLK, NUM_EXPERTS), jnp.float32)],
        compiler_params=pltpu.CompilerParams(
            dimension_semantics=("parallel", "arbitrary"),
        ),
    )(x_bf, tid_2d, w1_top, table_proj, w2_bf, b2_2d)

    return (indices, weights)


# speedup vs baseline: 1.3676x; 1.0233x over previous
import jax
import jax.numpy as jnp
from jax.experimental import pallas as pl
from jax.experimental.pallas import tpu as pltpu

D = 4096
NUM_TASKS = 16
NUM_EXPERTS = 64
TOP_K = 8
TOKENS = 8192

T_BLK = 512
N_BLK = 512


def _proj_kernel(table_ref, w1b_ref, b1_ref, out_ref):
    out_ref[...] = (
        jnp.dot(table_ref[...], w1b_ref[...], preferred_element_type=jnp.float32)
        + b1_ref[...]
    )


def _router_kernel(x_ref, tid_ref, w1t_ref, tp_ref, w2_ref, b2_ref,
                   idx_ref, w_ref):
    h = jnp.dot(x_ref[...], w1t_ref[...], preferred_element_type=jnp.float32)
    oh = (tid_ref[...] == jax.lax.broadcasted_iota(
        jnp.int32, (T_BLK, NUM_TASKS), 1)).astype(jnp.float32)
    h = h + jnp.dot(oh, tp_ref[...], preferred_element_type=jnp.float32,
                    precision=jax.lax.Precision.HIGHEST)
    h = jnp.maximum(h, 0.0)
    logits = jnp.dot(h.astype(jnp.bfloat16), w2_ref[...],
                     preferred_element_type=jnp.float32) + b2_ref[...]

    iota = jax.lax.broadcasted_iota(jnp.int32, (T_BLK, NUM_EXPERTS), 1)
    iota_f = iota.astype(jnp.float32)
    vals = logits
    neg_inf = jnp.float32(-jnp.inf)
    big = jnp.float32(NUM_EXPERTS)
    top_v = []
    top_i = []
    for _k in range(TOP_K):
        m = jnp.max(vals, axis=1, keepdims=True)
        is_max = vals == m
        idx_f = jnp.min(jnp.where(is_max, iota_f, big), axis=1, keepdims=True)
        top_v.append(m)
        top_i.append(idx_f)
        vals = jnp.where(iota_f == idx_f, neg_inf, vals)
    tv = jnp.concatenate(top_v, axis=1)
    ti = jnp.concatenate(top_i, axis=1)
    e = jnp.exp(tv - tv[:, 0:1])
    w = e / jnp.sum(e, axis=1, keepdims=True)
    idx_ref[...] = ti.astype(jnp.int32)
    w_ref[...] = w


def kernel(x, task_id, table, W1, b1, W2, b2):
    x_bf = x.astype(jnp.bfloat16)
    w1_top = W1[:D].astype(jnp.bfloat16)
    w1_bot = W1[D:].astype(jnp.bfloat16)
    table_bf = table.astype(jnp.bfloat16)
    w2_bf = W2.astype(jnp.bfloat16)
    b1_2d = b1.reshape(1, D)
    b2_2d = b2.reshape(1, NUM_EXPERTS)
    tid_2d = task_id.astype(jnp.int32).reshape(TOKENS, 1)

    n_nb = D // N_BLK
    table_proj = pl.pallas_call(
        _proj_kernel,
        grid=(n_nb,),
        in_specs=[
            pl.BlockSpec((NUM_TASKS, D), lambda n: (0, 0)),
            pl.BlockSpec((D, N_BLK), lambda n: (0, n)),
            pl.BlockSpec((1, N_BLK), lambda n: (0, n)),
        ],
        out_specs=pl.BlockSpec((NUM_TASKS, N_BLK), lambda n: (0, n)),
        out_shape=jax.ShapeDtypeStruct((NUM_TASKS, D), jnp.float32),
    )(table_bf, w1_bot, b1_2d)

    n_tb = TOKENS // T_BLK
    indices, weights = pl.pallas_call(
        _router_kernel,
        grid=(n_tb,),
        in_specs=[
            pl.BlockSpec((T_BLK, D), lambda t: (t, 0)),
            pl.BlockSpec((T_BLK, 1), lambda t: (t, 0)),
            pl.BlockSpec((D, D), lambda t: (0, 0)),
            pl.BlockSpec((NUM_TASKS, D), lambda t: (0, 0)),
            pl.BlockSpec((D, NUM_EXPERTS), lambda t: (0, 0)),
            pl.BlockSpec((1, NUM_EXPERTS), lambda t: (0, 0)),
        ],
        out_specs=[
            pl.BlockSpec((T_BLK, TOP_K), lambda t: (t, 0)),
            pl.BlockSpec((T_BLK, TOP_K), lambda t: (t, 0)),
        ],
        out_shape=[
            jax.ShapeDtypeStruct((TOKENS, TOP_K), jnp.int32),
            jax.ShapeDtypeStruct((TOKENS, TOP_K), jnp.float32),
        ],
        compiler_params=pltpu.CompilerParams(
            dimension_semantics=("arbitrary",),
        ),
    )(x_bf, tid_2d, w1_top, table_proj, w2_bf, b2_2d)

    return (indices, weights)


# speedup vs baseline: 1.4002x; 1.0238x over previous
import jax
import jax.numpy as jnp
from jax.experimental import pallas as pl
from jax.experimental.pallas import tpu as pltpu

D = 4096
NUM_TASKS = 16
NUM_EXPERTS = 64
TOP_K = 8
TOKENS = 8192

T_BLK = 256
C_BLK = 512
N_BLK = 512


def _cast_kernel(w_ref, out_ref):
    out_ref[...] = w_ref[...].astype(jnp.bfloat16)


def _proj_kernel(table_ref, w1b_ref, b1_ref, out_ref):
    out_ref[...] = (
        jnp.dot(table_ref[...].astype(jnp.bfloat16),
                w1b_ref[...].astype(jnp.bfloat16),
                preferred_element_type=jnp.float32)
        + b1_ref[...]
    )


def _router_kernel(x_ref, tid_ref, w1t_ref, tp_ref, w2_ref, b2_ref,
                   idx_ref, w_ref):
    h = jnp.dot(x_ref[...].astype(jnp.bfloat16), w1t_ref[...],
                preferred_element_type=jnp.float32)
    oh = (tid_ref[...] == jax.lax.broadcasted_iota(
        jnp.int32, (T_BLK, NUM_TASKS), 1)).astype(jnp.float32)
    h = h + jnp.dot(oh, tp_ref[...], preferred_element_type=jnp.float32,
                    precision=jax.lax.Precision.HIGHEST)
    h = jnp.maximum(h, 0.0)
    logits = jnp.dot(h.astype(jnp.bfloat16), w2_ref[...].astype(jnp.bfloat16),
                     preferred_element_type=jnp.float32) + b2_ref[...]

    iota = jax.lax.broadcasted_iota(jnp.int32, (T_BLK, NUM_EXPERTS), 1)
    iota_f = iota.astype(jnp.float32)
    vals = logits
    neg_inf = jnp.float32(-jnp.inf)
    big = jnp.float32(NUM_EXPERTS)
    top_v = []
    top_i = []
    for _k in range(TOP_K):
        m = jnp.max(vals, axis=1, keepdims=True)
        is_max = vals == m
        idx_f = jnp.min(jnp.where(is_max, iota_f, big), axis=1, keepdims=True)
        top_v.append(m)
        top_i.append(idx_f)
        vals = jnp.where(iota_f == idx_f, neg_inf, vals)
    tv = jnp.concatenate(top_v, axis=1)
    ti = jnp.concatenate(top_i, axis=1)
    e = jnp.exp(tv - tv[:, 0:1])
    w = e / jnp.sum(e, axis=1, keepdims=True)
    idx_ref[...] = ti.astype(jnp.int32)
    w_ref[...] = w


def kernel(x, task_id, table, W1, b1, W2, b2):
    w1_top = W1[:D]
    w1_bot = W1[D:]
    b1_2d = b1.reshape(1, D)
    b2_2d = b2.reshape(1, NUM_EXPERTS)
    tid_2d = task_id.astype(jnp.int32).reshape(TOKENS, 1)

    w1_top_bf = pl.pallas_call(
        _cast_kernel,
        grid=(D // C_BLK,),
        in_specs=[pl.BlockSpec((C_BLK, D), lambda i: (i, 0))],
        out_specs=pl.BlockSpec((C_BLK, D), lambda i: (i, 0)),
        out_shape=jax.ShapeDtypeStruct((D, D), jnp.bfloat16),
    )(w1_top)

    n_nb = D // N_BLK
    table_proj = pl.pallas_call(
        _proj_kernel,
        grid=(n_nb,),
        in_specs=[
            pl.BlockSpec((NUM_TASKS, D), lambda n: (0, 0)),
            pl.BlockSpec((D, N_BLK), lambda n: (0, n)),
            pl.BlockSpec((1, N_BLK), lambda n: (0, n)),
        ],
        out_specs=pl.BlockSpec((NUM_TASKS, N_BLK), lambda n: (0, n)),
        out_shape=jax.ShapeDtypeStruct((NUM_TASKS, D), jnp.float32),
    )(table, w1_bot, b1_2d)

    n_tb = TOKENS // T_BLK
    indices, weights = pl.pallas_call(
        _router_kernel,
        grid=(n_tb,),
        in_specs=[
            pl.BlockSpec((T_BLK, D), lambda t: (t, 0)),
            pl.BlockSpec((T_BLK, 1), lambda t: (t, 0)),
            pl.BlockSpec((D, D), lambda t: (0, 0)),
            pl.BlockSpec((NUM_TASKS, D), lambda t: (0, 0)),
            pl.BlockSpec((D, NUM_EXPERTS), lambda t: (0, 0)),
            pl.BlockSpec((1, NUM_EXPERTS), lambda t: (0, 0)),
        ],
        out_specs=[
            pl.BlockSpec((T_BLK, TOP_K), lambda t: (t, 0)),
            pl.BlockSpec((T_BLK, TOP_K), lambda t: (t, 0)),
        ],
        out_shape=[
            jax.ShapeDtypeStruct((TOKENS, TOP_K), jnp.int32),
            jax.ShapeDtypeStruct((TOKENS, TOP_K), jnp.float32),
        ],
        compiler_params=pltpu.CompilerParams(
            dimension_semantics=("arbitrary",),
        ),
    )(x, tid_2d, w1_top_bf, table_proj, W2, b2_2d)

    return (indices, weights)


# speedup vs baseline: 1.6077x; 1.1482x over previous
import jax
import jax.numpy as jnp
from jax.experimental import pallas as pl
from jax.experimental.pallas import tpu as pltpu

D = 4096
NUM_TASKS = 16
NUM_EXPERTS = 64
TOP_K = 8
TOKENS = 8192

T_BLK = 512
H_BLK = 512
C_BLK = 512
N_BLK = 512


def _cast_kernel(w_ref, out_ref):
    out_ref[...] = w_ref[...].astype(jnp.bfloat16)


def _proj_kernel(table_ref, w1b_ref, b1_ref, out_ref):
    tp = (
        jnp.dot(table_ref[...].astype(jnp.bfloat16),
                w1b_ref[...].astype(jnp.bfloat16),
                preferred_element_type=jnp.float32)
        + b1_ref[...]
    )
    p1 = tp.astype(jnp.bfloat16)
    r1 = tp - p1.astype(jnp.float32)
    p2 = r1.astype(jnp.bfloat16)
    r2 = r1 - p2.astype(jnp.float32)
    p3 = r2.astype(jnp.bfloat16)
    out_ref[...] = jnp.concatenate([p1, p2, p3], axis=0)


def _router_kernel(x_ref, tid_ref, w1t_ref, tp_ref, w2_ref, b2_ref,
                   idx_ref, w_ref):
    oh3 = ((jax.lax.broadcasted_iota(jnp.int32, (T_BLK, 3 * NUM_TASKS), 1)
            & (NUM_TASKS - 1)) == tid_ref[...]).astype(jnp.bfloat16)
    x_bf = x_ref[...].astype(jnp.bfloat16)
    w2_bf = w2_ref[...].astype(jnp.bfloat16)
    logits = b2_ref[...]
    for c in range(D // H_BLK):
        lo, hi = c * H_BLK, (c + 1) * H_BLK
        hc = jnp.dot(x_bf, w1t_ref[:, lo:hi], preferred_element_type=jnp.float32)
        hc = hc + jnp.dot(oh3, tp_ref[:, lo:hi], preferred_element_type=jnp.float32)
        hc = jnp.maximum(hc, 0.0)
        logits = logits + jnp.dot(hc.astype(jnp.bfloat16), w2_bf[lo:hi, :],
                                  preferred_element_type=jnp.float32)

    iota = jax.lax.broadcasted_iota(jnp.int32, (T_BLK, NUM_EXPERTS), 1)
    iota_f = iota.astype(jnp.float32)
    vals = logits
    neg_inf = jnp.float32(-jnp.inf)
    big = jnp.float32(NUM_EXPERTS)
    top_v = []
    top_i = []
    for _k in range(TOP_K):
        m = jnp.max(vals, axis=1, keepdims=True)
        is_max = vals == m
        idx_f = jnp.min(jnp.where(is_max, iota_f, big), axis=1, keepdims=True)
        top_v.append(m)
        top_i.append(idx_f)
        vals = jnp.where(iota_f == idx_f, neg_inf, vals)
    tv = jnp.concatenate(top_v, axis=1)
    ti = jnp.concatenate(top_i, axis=1)
    e = jnp.exp(tv - tv[:, 0:1])
    w = e / jnp.sum(e, axis=1, keepdims=True)
    idx_ref[...] = ti.astype(jnp.int32)
    w_ref[...] = w


def kernel(x, task_id, table, W1, b1, W2, b2):
    w1_top = W1[:D]
    w1_bot = W1[D:]
    b1_2d = b1.reshape(1, D)
    b2_2d = b2.reshape(1, NUM_EXPERTS)
    tid_2d = task_id.astype(jnp.int32).reshape(TOKENS, 1)

    w1_top_bf = pl.pallas_call(
        _cast_kernel,
        grid=(D // C_BLK,),
        in_specs=[pl.BlockSpec((C_BLK, D), lambda i: (i, 0))],
        out_specs=pl.BlockSpec((C_BLK, D), lambda i: (i, 0)),
        out_shape=jax.ShapeDtypeStruct((D, D), jnp.bfloat16),
    )(w1_top)

    n_nb = D // N_BLK
    table_proj = pl.pallas_call(
        _proj_kernel,
        grid=(n_nb,),
        in_specs=[
            pl.BlockSpec((NUM_TASKS, D), lambda n: (0, 0)),
            pl.BlockSpec((D, N_BLK), lambda n: (0, n)),
            pl.BlockSpec((1, N_BLK), lambda n: (0, n)),
        ],
        out_specs=pl.BlockSpec((3 * NUM_TASKS, N_BLK), lambda n: (0, n)),
        out_shape=jax.ShapeDtypeStruct((3 * NUM_TASKS, D), jnp.bfloat16),
    )(table, w1_bot, b1_2d)

    n_tb = TOKENS // T_BLK
    indices, weights = pl.pallas_call(
        _router_kernel,
        grid=(n_tb,),
        in_specs=[
            pl.BlockSpec((T_BLK, D), lambda t: (t, 0)),
            pl.BlockSpec((T_BLK, 1), lambda t: (t, 0)),
            pl.BlockSpec((D, D), lambda t: (0, 0)),
            pl.BlockSpec((3 * NUM_TASKS, D), lambda t: (0, 0)),
            pl.BlockSpec((D, NUM_EXPERTS), lambda t: (0, 0)),
            pl.BlockSpec((1, NUM_EXPERTS), lambda t: (0, 0)),
        ],
        out_specs=[
            pl.BlockSpec((T_BLK, TOP_K), lambda t: (t, 0)),
            pl.BlockSpec((T_BLK, TOP_K), lambda t: (t, 0)),
        ],
        out_shape=[
            jax.ShapeDtypeStruct((TOKENS, TOP_K), jnp.int32),
            jax.ShapeDtypeStruct((TOKENS, TOP_K), jnp.float32),
        ],
        compiler_params=pltpu.CompilerParams(
            dimension_semantics=("arbitrary",),
        ),
    )(x, tid_2d, w1_top_bf, table_proj, W2, b2_2d)

    return (indices, weights)


# speedup vs baseline: 1.6250x; 1.0107x over previous
import jax
import jax.numpy as jnp
from jax.experimental import pallas as pl
from jax.experimental.pallas import tpu as pltpu

D = 4096
NUM_TASKS = 16
NUM_EXPERTS = 64
TOP_K = 8
TOKENS = 8192

T_BLK = 512
H_BLK = 512
C_BLK = 512
N_BLK = 512


def _proj_kernel(table_ref, w1b_ref, b1_ref, w1t_ref, out_ref, w1t_bf_ref):
    w1t_bf_ref[...] = w1t_ref[...].astype(jnp.bfloat16)
    tp = (
        jnp.dot(table_ref[...].astype(jnp.bfloat16),
                w1b_ref[...].astype(jnp.bfloat16),
                preferred_element_type=jnp.float32)
        + b1_ref[...]
    )
    p1 = tp.astype(jnp.bfloat16)
    r1 = tp - p1.astype(jnp.float32)
    p2 = r1.astype(jnp.bfloat16)
    r2 = r1 - p2.astype(jnp.float32)
    p3 = r2.astype(jnp.bfloat16)
    out_ref[...] = jnp.concatenate([p1, p2, p3], axis=0)


def _router_kernel(x_ref, tid_ref, w1t_ref, tp_ref, w2_ref, b2_ref,
                   idx_ref, w_ref):
    oh3 = ((jax.lax.broadcasted_iota(jnp.int32, (T_BLK, 3 * NUM_TASKS), 1)
            & (NUM_TASKS - 1)) == tid_ref[...]).astype(jnp.bfloat16)
    x_bf = x_ref[...].astype(jnp.bfloat16)
    w2_bf = w2_ref[...].astype(jnp.bfloat16)
    logits = b2_ref[...]
    for c in range(D // H_BLK):
        lo, hi = c * H_BLK, (c + 1) * H_BLK
        hc = jnp.dot(x_bf, w1t_ref[:, lo:hi], preferred_element_type=jnp.float32)
        hc = hc + jnp.dot(oh3, tp_ref[:, lo:hi], preferred_element_type=jnp.float32)
        hc = jnp.maximum(hc, 0.0)
        logits = logits + jnp.dot(hc.astype(jnp.bfloat16), w2_bf[lo:hi, :],
                                  preferred_element_type=jnp.float32)

    iota = jax.lax.broadcasted_iota(jnp.int32, (T_BLK, NUM_EXPERTS), 1)
    iota_f = iota.astype(jnp.float32)
    vals = logits
    neg_inf = jnp.float32(-jnp.inf)
    big = jnp.float32(NUM_EXPERTS)
    top_v = []
    top_i = []
    for _k in range(TOP_K):
        m = jnp.max(vals, axis=1, keepdims=True)
        is_max = vals == m
        idx_f = jnp.min(jnp.where(is_max, iota_f, big), axis=1, keepdims=True)
        top_v.append(m)
        top_i.append(idx_f)
        vals = jnp.where(iota_f == idx_f, neg_inf, vals)
    tv = jnp.concatenate(top_v, axis=1)
    ti = jnp.concatenate(top_i, axis=1)
    e = jnp.exp(tv - tv[:, 0:1])
    w = e / jnp.sum(e, axis=1, keepdims=True)
    idx_ref[...] = ti.astype(jnp.int32)
    w_ref[...] = w


def kernel(x, task_id, table, W1, b1, W2, b2):
    w1_top = W1[:D]
    w1_bot = W1[D:]
    b1_2d = b1.reshape(1, D)
    b2_2d = b2.reshape(1, NUM_EXPERTS)
    tid_2d = task_id.astype(jnp.int32).reshape(TOKENS, 1)

    n_nb = D // N_BLK
    table_proj, w1_top_bf = pl.pallas_call(
        _proj_kernel,
        grid=(n_nb,),
        in_specs=[
            pl.BlockSpec((NUM_TASKS, D), lambda n: (0, 0)),
            pl.BlockSpec((D, N_BLK), lambda n: (0, n)),
            pl.BlockSpec((1, N_BLK), lambda n: (0, n)),
            pl.BlockSpec((D, N_BLK), lambda n: (0, n)),
        ],
        out_specs=[
            pl.BlockSpec((3 * NUM_TASKS, N_BLK), lambda n: (0, n)),
            pl.BlockSpec((D, N_BLK), lambda n: (0, n)),
        ],
        out_shape=[
            jax.ShapeDtypeStruct((3 * NUM_TASKS, D), jnp.bfloat16),
            jax.ShapeDtypeStruct((D, D), jnp.bfloat16),
        ],
    )(table, w1_bot, b1_2d, w1_top)

    n_tb = TOKENS // T_BLK
    indices, weights = pl.pallas_call(
        _router_kernel,
        grid=(n_tb,),
        in_specs=[
            pl.BlockSpec((T_BLK, D), lambda t: (t, 0)),
            pl.BlockSpec((T_BLK, 1), lambda t: (t, 0)),
            pl.BlockSpec((D, D), lambda t: (0, 0)),
            pl.BlockSpec((3 * NUM_TASKS, D), lambda t: (0, 0)),
            pl.BlockSpec((D, NUM_EXPERTS), lambda t: (0, 0)),
            pl.BlockSpec((1, NUM_EXPERTS), lambda t: (0, 0)),
        ],
        out_specs=[
            pl.BlockSpec((T_BLK, TOP_K), lambda t: (t, 0)),
            pl.BlockSpec((T_BLK, TOP_K), lambda t: (t, 0)),
        ],
        out_shape=[
            jax.ShapeDtypeStruct((TOKENS, TOP_K), jnp.int32),
            jax.ShapeDtypeStruct((TOKENS, TOP_K), jnp.float32),
        ],
        compiler_params=pltpu.CompilerParams(
            dimension_semantics=("parallel",),
        ),
    )(x, tid_2d, w1_top_bf, table_proj, W2, b2_2d)

    return (indices, weights)


# speedup vs baseline: 1.6281x; 1.0019x over previous
import jax
import jax.numpy as jnp
from jax.experimental import pallas as pl
from jax.experimental.pallas import tpu as pltpu

D = 4096
NUM_TASKS = 16
NUM_EXPERTS = 64
TOP_K = 8
TOKENS = 8192

T_BLK = 256
H_BLK = 1024
C_BLK = 512
N_BLK = 512


def _proj_kernel(table_ref, w1b_ref, b1_ref, w1t_ref, out_ref, w1t_bf_ref):
    w1t_bf_ref[...] = w1t_ref[...].astype(jnp.bfloat16)
    tp = (
        jnp.dot(table_ref[...].astype(jnp.bfloat16),
                w1b_ref[...].astype(jnp.bfloat16),
                preferred_element_type=jnp.float32)
        + b1_ref[...]
    )
    p1 = tp.astype(jnp.bfloat16)
    r1 = tp - p1.astype(jnp.float32)
    p2 = r1.astype(jnp.bfloat16)
    r2 = r1 - p2.astype(jnp.float32)
    p3 = r2.astype(jnp.bfloat16)
    out_ref[...] = jnp.concatenate([p1, p2, p3], axis=0)


def _router_kernel(x_ref, tid_ref, w1t_ref, tp_ref, w2_ref, b2_ref,
                   idx_ref, w_ref):
    oh3 = ((jax.lax.broadcasted_iota(jnp.int32, (T_BLK, 3 * NUM_TASKS), 1)
            & (NUM_TASKS - 1)) == tid_ref[...]).astype(jnp.bfloat16)
    x_bf = x_ref[...].astype(jnp.bfloat16)
    w2_bf = w2_ref[...].astype(jnp.bfloat16)
    logits = b2_ref[...]
    for c in range(D // H_BLK):
        lo, hi = c * H_BLK, (c + 1) * H_BLK
        hc = jnp.dot(x_bf, w1t_ref[:, lo:hi], preferred_element_type=jnp.float32)
        hc = hc + jnp.dot(oh3, tp_ref[:, lo:hi], preferred_element_type=jnp.float32)
        hc = jnp.maximum(hc, 0.0)
        logits = logits + jnp.dot(hc.astype(jnp.bfloat16), w2_bf[lo:hi, :],
                                  preferred_element_type=jnp.float32)

    iota = jax.lax.broadcasted_iota(jnp.int32, (T_BLK, NUM_EXPERTS), 1)
    iota_f = iota.astype(jnp.float32)
    vals = logits
    neg_inf = jnp.float32(-jnp.inf)
    big = jnp.float32(NUM_EXPERTS)
    top_v = []
    top_i = []
    for _k in range(TOP_K):
        m = jnp.max(vals, axis=1, keepdims=True)
        is_max = vals == m
        idx_f = jnp.min(jnp.where(is_max, iota_f, big), axis=1, keepdims=True)
        top_v.append(m)
        top_i.append(idx_f)
        vals = jnp.where(iota_f == idx_f, neg_inf, vals)
    tv = jnp.concatenate(top_v, axis=1)
    ti = jnp.concatenate(top_i, axis=1)
    e = jnp.exp(tv - tv[:, 0:1])
    w = e / jnp.sum(e, axis=1, keepdims=True)
    idx_ref[...] = ti.astype(jnp.int32)
    w_ref[...] = w


def kernel(x, task_id, table, W1, b1, W2, b2):
    w1_top = W1[:D]
    w1_bot = W1[D:]
    b1_2d = b1.reshape(1, D)
    b2_2d = b2.reshape(1, NUM_EXPERTS)
    tid_2d = task_id.astype(jnp.int32).reshape(TOKENS, 1)

    n_nb = D // N_BLK
    table_proj, w1_top_bf = pl.pallas_call(
        _proj_kernel,
        grid=(n_nb,),
        in_specs=[
            pl.BlockSpec((NUM_TASKS, D), lambda n: (0, 0)),
            pl.BlockSpec((D, N_BLK), lambda n: (0, n)),
            pl.BlockSpec((1, N_BLK), lambda n: (0, n)),
            pl.BlockSpec((D, N_BLK), lambda n: (0, n)),
        ],
        out_specs=[
            pl.BlockSpec((3 * NUM_TASKS, N_BLK), lambda n: (0, n)),
            pl.BlockSpec((D, N_BLK), lambda n: (0, n)),
        ],
        out_shape=[
            jax.ShapeDtypeStruct((3 * NUM_TASKS, D), jnp.bfloat16),
            jax.ShapeDtypeStruct((D, D), jnp.bfloat16),
        ],
    )(table, w1_bot, b1_2d, w1_top)

    n_tb = TOKENS // T_BLK
    indices, weights = pl.pallas_call(
        _router_kernel,
        grid=(n_tb,),
        in_specs=[
            pl.BlockSpec((T_BLK, D), lambda t: (t, 0)),
            pl.BlockSpec((T_BLK, 1), lambda t: (t, 0)),
            pl.BlockSpec((D, D), lambda t: (0, 0)),
            pl.BlockSpec((3 * NUM_TASKS, D), lambda t: (0, 0)),
            pl.BlockSpec((D, NUM_EXPERTS), lambda t: (0, 0)),
            pl.BlockSpec((1, NUM_EXPERTS), lambda t: (0, 0)),
        ],
        out_specs=[
            pl.BlockSpec((T_BLK, TOP_K), lambda t: (t, 0)),
            pl.BlockSpec((T_BLK, TOP_K), lambda t: (t, 0)),
        ],
        out_shape=[
            jax.ShapeDtypeStruct((TOKENS, TOP_K), jnp.int32),
            jax.ShapeDtypeStruct((TOKENS, TOP_K), jnp.float32),
        ],
        compiler_params=pltpu.CompilerParams(
            dimension_semantics=("parallel",),
        ),
    )(x, tid_2d, w1_top_bf, table_proj, W2, b2_2d)

    return (indices, weights)


# speedup vs baseline: 1.6403x; 1.0075x over previous
import jax
import jax.numpy as jnp
from jax.experimental import pallas as pl
from jax.experimental.pallas import tpu as pltpu

D = 4096
NUM_TASKS = 16
NUM_EXPERTS = 64
TOP_K = 8
TOKENS = 8192

T_BLK = 256
H_BLK = 2048
C_BLK = 512
N_BLK = 512


def _proj_kernel(table_ref, w1b_ref, b1_ref, w1t_ref, out_ref, w1t_bf_ref):
    w1t_bf_ref[...] = w1t_ref[...].astype(jnp.bfloat16)
    tp = (
        jnp.dot(table_ref[...].astype(jnp.bfloat16),
                w1b_ref[...].astype(jnp.bfloat16),
                preferred_element_type=jnp.float32)
        + b1_ref[...]
    )
    p1 = tp.astype(jnp.bfloat16)
    r1 = tp - p1.astype(jnp.float32)
    p2 = r1.astype(jnp.bfloat16)
    r2 = r1 - p2.astype(jnp.float32)
    p3 = r2.astype(jnp.bfloat16)
    out_ref[...] = jnp.concatenate([p1, p2, p3], axis=0)


def _router_kernel(x_ref, tid_ref, w1t_ref, tp_ref, w2_ref, b2_ref,
                   idx_ref, w_ref):
    oh3 = ((jax.lax.broadcasted_iota(jnp.int32, (T_BLK, 3 * NUM_TASKS), 1)
            & (NUM_TASKS - 1)) == tid_ref[...]).astype(jnp.bfloat16)
    x_bf = x_ref[...].astype(jnp.bfloat16)
    w2_bf = w2_ref[...].astype(jnp.bfloat16)
    logits = b2_ref[...]
    for c in range(D // H_BLK):
        lo, hi = c * H_BLK, (c + 1) * H_BLK
        hc = jnp.dot(x_bf, w1t_ref[:, lo:hi], preferred_element_type=jnp.float32)
        hc = hc + jnp.dot(oh3, tp_ref[:, lo:hi], preferred_element_type=jnp.float32)
        hc = jnp.maximum(hc, 0.0)
        logits = logits + jnp.dot(hc.astype(jnp.bfloat16), w2_bf[lo:hi, :],
                                  preferred_element_type=jnp.float32)

    iota = jax.lax.broadcasted_iota(jnp.int32, (T_BLK, NUM_EXPERTS), 1)
    iota_f = iota.astype(jnp.float32)
    vals = logits
    neg_inf = jnp.float32(-jnp.inf)
    big = jnp.float32(NUM_EXPERTS)
    top_v = []
    top_i = []
    for _k in range(TOP_K):
        m = jnp.max(vals, axis=1, keepdims=True)
        is_max = vals == m
        idx_f = jnp.min(jnp.where(is_max, iota_f, big), axis=1, keepdims=True)
        top_v.append(m)
        top_i.append(idx_f)
        vals = jnp.where(iota_f == idx_f, neg_inf, vals)
    tv = jnp.concatenate(top_v, axis=1)
    ti = jnp.concatenate(top_i, axis=1)
    e = jnp.exp(tv - tv[:, 0:1])
    w = e / jnp.sum(e, axis=1, keepdims=True)
    idx_ref[...] = ti.astype(jnp.int32)
    w_ref[...] = w


def kernel(x, task_id, table, W1, b1, W2, b2):
    w1_top = W1[:D]
    w1_bot = W1[D:]
    b1_2d = b1.reshape(1, D)
    b2_2d = b2.reshape(1, NUM_EXPERTS)
    tid_2d = task_id.astype(jnp.int32).reshape(TOKENS, 1)

    n_nb = D // N_BLK
    table_proj, w1_top_bf = pl.pallas_call(
        _proj_kernel,
        grid=(n_nb,),
        in_specs=[
            pl.BlockSpec((NUM_TASKS, D), lambda n: (0, 0)),
            pl.BlockSpec((D, N_BLK), lambda n: (0, n)),
            pl.BlockSpec((1, N_BLK), lambda n: (0, n)),
            pl.BlockSpec((D, N_BLK), lambda n: (0, n)),
        ],
        out_specs=[
            pl.BlockSpec((3 * NUM_TASKS, N_BLK), lambda n: (0, n)),
            pl.BlockSpec((D, N_BLK), lambda n: (0, n)),
        ],
        out_shape=[
            jax.ShapeDtypeStruct((3 * NUM_TASKS, D), jnp.bfloat16),
            jax.ShapeDtypeStruct((D, D), jnp.bfloat16),
        ],
    )(table, w1_bot, b1_2d, w1_top)

    n_tb = TOKENS // T_BLK
    indices, weights = pl.pallas_call(
        _router_kernel,
        grid=(n_tb,),
        in_specs=[
            pl.BlockSpec((T_BLK, D), lambda t: (t, 0)),
            pl.BlockSpec((T_BLK, 1), lambda t: (t, 0)),
            pl.BlockSpec((D, D), lambda t: (0, 0)),
            pl.BlockSpec((3 * NUM_TASKS, D), lambda t: (0, 0)),
            pl.BlockSpec((D, NUM_EXPERTS), lambda t: (0, 0)),
            pl.BlockSpec((1, NUM_EXPERTS), lambda t: (0, 0)),
        ],
        out_specs=[
            pl.BlockSpec((T_BLK, TOP_K), lambda t: (t, 0)),
            pl.BlockSpec((T_BLK, TOP_K), lambda t: (t, 0)),
        ],
        out_shape=[
            jax.ShapeDtypeStruct((TOKENS, TOP_K), jnp.int32),
            jax.ShapeDtypeStruct((TOKENS, TOP_K), jnp.float32),
        ],
        compiler_params=pltpu.CompilerParams(
            dimension_semantics=("parallel",),
        ),
    )(x, tid_2d, w1_top_bf, table_proj, W2, b2_2d)

    return (indices, weights)
